# baseline (speedup 1.0000x reference)
import jax
import jax.numpy as jnp
from jax.experimental import pallas as pl

_NV = 8192
_D = 32
_BETA = 0.25
_T = 256
_WIN = 2048


def _vq_body(z_ref, inp_ref, w_ref, zq_ref, idx_ref, loss_ref):
    i = pl.program_id(0)
    zt = z_ref[...]
    w = w_ref[...]
    w2 = jnp.sum(w * w, axis=1)
    z2 = jnp.sum(zt * zt, axis=1)
    dot = jax.lax.dot_general(zt.astype(jnp.bfloat16), w.astype(jnp.bfloat16),
                              (((1,), (1,)), ((), ())),
                              preferred_element_type=jnp.float32)
    s = (z2[:, None] + w2[None, :]) - 2.0 * dot
    d = jnp.sqrt(jnp.maximum(s, 0.0))

    nw = _NV // _WIN
    best_v = jnp.full((_T,), jnp.inf, jnp.float32)
    best_i = jnp.zeros((_T,), jnp.int32)
    col = jax.lax.broadcasted_iota(jnp.int32, (_T, _WIN), 1)
    for wi in range(nw):
        dw = d[:, wi * _WIN:(wi + 1) * _WIN]
        mv = jnp.min(dw, axis=1)
        li = jnp.min(jnp.where(dw == mv[:, None], col, _WIN), axis=1) + wi * _WIN
        take = mv < best_v
        best_v = jnp.where(take, mv.astype(jnp.bfloat16).astype(jnp.float32),
                           best_v)
        best_i = jnp.where(take, li, best_i)
    idx_ref[0, 0, :] = best_i

    colf = jax.lax.broadcasted_iota(jnp.int32, (_T, _NV), 1)
    onehot = (colf == best_i[:, None]).astype(jnp.float32)
    zq = jax.lax.dot_general(onehot, w, (((1,), (0,)), ((), ())),
                             preferred_element_type=jnp.float32)
    zq_ref[...] = zq
    diff = zq - inp_ref[...]
    part = jnp.sum(diff * diff).reshape(1, 1)

    @pl.when(i == 0)
    def _():
        loss_ref[...] = part

    @pl.when(i != 0)
    def _():
        loss_ref[...] += part


def kernel(input, W):
    B, D, H, Wd = input.shape
    N = B * H * Wd
    z = jnp.transpose(input, (0, 2, 3, 1)).reshape(N, D)
    inp2 = input.reshape(N, D)
    G = N // _T
    zq_flat, idx3, loss_sum = pl.pallas_call(
        _vq_body,
        grid=(G,),
        in_specs=[
            pl.BlockSpec((_T, D), lambda i: (i, 0)),
            pl.BlockSpec((_T, D), lambda i: (i, 0)),
            pl.BlockSpec((_NV, D), lambda i: (0, 0)),
        ],
        out_specs=[
            pl.BlockSpec((_T, D), lambda i: (i, 0)),
            pl.BlockSpec((1, 1, _T), lambda i: (i, 0, 0)),
            pl.BlockSpec((1, 1), lambda i: (0, 0)),
        ],
        out_shape=[
            jax.ShapeDtypeStruct((N, D), jnp.float32),
            jax.ShapeDtypeStruct((G, 1, _T), jnp.int32),
            jax.ShapeDtypeStruct((1, 1), jnp.float32),
        ],
    )(z, inp2, W)
    idx = idx3.reshape(N)
    z_q = zq_flat.reshape(input.shape)
    loss = loss_sum[0, 0] * ((1.0 + _BETA) / (N * D))
    return (z_q, idx, loss)


# speedup vs baseline: 1.2044x; 1.2044x over previous
import jax
import jax.numpy as jnp
from jax import lax
from jax.experimental import pallas as pl
from jax.experimental.pallas import tpu as pltpu, tpu_sc as plsc

_NV = 8192
_D = 32
_BETA = 0.25
_T = 256
_WIN = 2048
_N = 8192

_info = plsc.get_sparse_core_info()
_NC, _NS, _L = _info.num_cores, _info.num_subcores, _info.num_lanes
_NW = _NC * _NS
_BPW = _N // _NW


def _argmin_body(z_ref, w_ref, z2_ref, w2_ref, idx_ref):
    zt = z_ref[...]
    w = w_ref[...]
    z2 = z2_ref[0, 0, :]
    w2 = w2_ref[0, :]
    dot = jax.lax.dot_general(zt.astype(jnp.bfloat16), w.astype(jnp.bfloat16),
                              (((1,), (1,)), ((), ())),
                              preferred_element_type=jnp.float32)
    s = (z2[:, None] + w2[None, :]) - 2.0 * dot
    d = jnp.sqrt(jnp.maximum(s, 0.0))

    nw = _NV // _WIN
    best_v = jnp.full((_T,), jnp.inf, jnp.float32)
    best_i = jnp.zeros((_T,), jnp.int32)
    col = jax.lax.broadcasted_iota(jnp.int32, (_T, _WIN), 1)
    for wi in range(nw):
        dw = d[:, wi * _WIN:(wi + 1) * _WIN]
        mv = jnp.min(dw, axis=1)
        li = jnp.min(jnp.where(dw == mv[:, None], col, _WIN), axis=1) + wi * _WIN
        take = mv < best_v
        mb = jax.lax.bitcast_convert_type(mv, jnp.uint32)
        mb = (mb + 0x7FFF + ((mb >> 16) & 1)) & jnp.uint32(0xFFFF0000)
        mvr = jax.lax.bitcast_convert_type(mb, jnp.float32)
        best_v = jnp.where(take, mvr, best_v)
        best_i = jnp.where(take, li, best_i)
    idx_ref[0, 0, :] = best_i


def _sc_gather_loss(w_hbm, idx_hbm, inp_hbm, zq_hbm, part_hbm,
                    idx_v, rows_v, inp_v, zqc_v, part_v, sem):
    wid = lax.axis_index("s") * _NC + lax.axis_index("c")
    base = wid * _BPW
    pltpu.sync_copy(idx_hbm.at[pl.ds(base, _BPW)], idx_v)
    pltpu.async_copy(w_hbm.at[idx_v], rows_v, sem).wait()
    pltpu.sync_copy(inp_hbm.at[pl.ds(base, _BPW)], inp_v)

    def body(t, acc):
        a0 = rows_v[t, pl.ds(0, _L)]
        a1 = rows_v[t, pl.ds(_L, _L)]
        zqc_v[t, pl.ds(0, _L)] = a0
        zqc_v[t, pl.ds(_L, _L)] = a1
        e0 = a0 - inp_v[t, pl.ds(0, _L)]
        e1 = a1 - inp_v[t, pl.ds(_L, _L)]
        return acc + e0 * e0 + e1 * e1

    acc = lax.fori_loop(0, _BPW, body, jnp.zeros((_L,), jnp.float32))
    pltpu.sync_copy(zqc_v, zq_hbm.at[pl.ds(base, _BPW)])
    part_v[...] = acc
    pltpu.sync_copy(part_v, part_hbm.at[wid])


def kernel(input, W):
    B, D, H, Wd = input.shape
    N = B * H * Wd
    z = jnp.transpose(input, (0, 2, 3, 1)).reshape(N, D)
    inp2 = input.reshape(N, D)
    G = N // _T
    z2 = jnp.sum(z * z, axis=1).reshape(G, 1, _T)
    w2 = jnp.sum(W * W, axis=1).reshape(1, _NV)
    idx3 = pl.pallas_call(
        _argmin_body,
        grid=(G,),
        in_specs=[
            pl.BlockSpec((_T, D), lambda i: (i, 0)),
            pl.BlockSpec((_NV, D), lambda i: (0, 0)),
            pl.BlockSpec((1, 1, _T), lambda i: (i, 0, 0)),
            pl.BlockSpec((1, _NV), lambda i: (0, 0)),
        ],
        out_specs=pl.BlockSpec((1, 1, _T), lambda i: (i, 0, 0)),
        out_shape=jax.ShapeDtypeStruct((G, 1, _T), jnp.int32),
    )(z, W, z2, w2)
    idx = idx3.reshape(N)

    w_pad = jnp.pad(W, ((0, 0), (0, 128 - D)))
    mesh = plsc.VectorSubcoreMesh(core_axis_name="c", subcore_axis_name="s")
    zq_flat, partials = pl.kernel(
        _sc_gather_loss,
        mesh=mesh,
        out_type=[
            jax.ShapeDtypeStruct((N, D), jnp.float32),
            jax.ShapeDtypeStruct((_NW, _L), jnp.float32),
        ],
        scratch_types=[
            pltpu.VMEM((_BPW,), jnp.int32),
            pltpu.VMEM((_BPW, 128), jnp.float32),
            pltpu.VMEM((_BPW, _D), jnp.float32),
            pltpu.VMEM((_BPW, _D), jnp.float32),
            pltpu.VMEM((_L,), jnp.float32),
            pltpu.SemaphoreType.DMA,
        ],
    )(w_pad, idx, inp2)

    z_q = zq_flat.reshape(input.shape)
    loss = jnp.sum(partials) * ((1.0 + _BETA) / (N * D))
    return (z_q, idx, loss)


# speedup vs baseline: 1.2780x; 1.0611x over previous
import jax
import jax.numpy as jnp
from jax import lax
from jax.experimental import pallas as pl
from jax.experimental.pallas import tpu as pltpu, tpu_sc as plsc

_NV = 8192
_D = 32
_BETA = 0.25
_T = 512
_WIN = 2048
_N = 8192

_info = plsc.get_sparse_core_info()
_NC, _NS, _L = _info.num_cores, _info.num_subcores, _info.num_lanes
_NW = _NC * _NS
_BPW = _N // _NW


def _argmin_body(z_ref, w_ref, z2_ref, w2_ref, idx_ref):
    zt = z_ref[...]
    w = w_ref[...]
    z2 = z2_ref[0, 0, :]
    w2 = w2_ref[0, :]
    dot = jax.lax.dot_general(zt.astype(jnp.bfloat16), w.astype(jnp.bfloat16),
                              (((1,), (1,)), ((), ())),
                              preferred_element_type=jnp.float32)
    s = (z2[:, None] + w2[None, :]) - 2.0 * dot
    d = jnp.sqrt(jnp.maximum(s, 0.0))

    nw = _NV // _WIN
    best_v = jnp.full((_T,), jnp.inf, jnp.float32)
    best_i = jnp.zeros((_T,), jnp.int32)
    col = jax.lax.broadcasted_iota(jnp.int32, (_T, _WIN), 1)
    for wi in range(nw):
        dw = d[:, wi * _WIN:(wi + 1) * _WIN]
        mv = jnp.min(dw, axis=1)
        li = jnp.min(jnp.where(dw == mv[:, None], col, _WIN), axis=1) + wi * _WIN
        take = mv < best_v
        mb = jax.lax.bitcast_convert_type(mv, jnp.uint32)
        mb = (mb + 0x7FFF + ((mb >> 16) & 1)) & jnp.uint32(0xFFFF0000)
        mvr = jax.lax.bitcast_convert_type(mb, jnp.float32)
        best_v = jnp.where(take, mvr, best_v)
        best_i = jnp.where(take, li, best_i)
    idx_ref[0, 0, :] = best_i


def _sc_gather_loss(w_hbm, idx_hbm, inp_hbm, zq_hbm, part_hbm,
                    idx_v, rows_v, inp_v, zqc_v, part_v, sem):
    wid = lax.axis_index("s") * _NC + lax.axis_index("c")
    base = wid * _BPW
    pltpu.sync_copy(idx_hbm.at[pl.ds(base, _BPW)], idx_v)
    pltpu.async_copy(w_hbm.at[idx_v], rows_v, sem).wait()
    pltpu.sync_copy(inp_hbm.at[pl.ds(base, _BPW)], inp_v)

    def body(t, acc):
        a0 = rows_v[t, pl.ds(0, _L)]
        a1 = rows_v[t, pl.ds(_L, _L)]
        zqc_v[t, pl.ds(0, _L)] = a0
        zqc_v[t, pl.ds(_L, _L)] = a1
        e0 = a0 - inp_v[t, pl.ds(0, _L)]
        e1 = a1 - inp_v[t, pl.ds(_L, _L)]
        return acc + e0 * e0 + e1 * e1

    acc = lax.fori_loop(0, _BPW, body, jnp.zeros((_L,), jnp.float32))
    pltpu.sync_copy(zqc_v, zq_hbm.at[pl.ds(base, _BPW)])
    part_v[...] = acc
    pltpu.sync_copy(part_v, part_hbm.at[wid])


def kernel(input, W):
    B, D, H, Wd = input.shape
    N = B * H * Wd
    z = jnp.transpose(input, (0, 2, 3, 1)).reshape(N, D)
    inp2 = input.reshape(N, D)
    G = N // _T
    z2 = jnp.sum(z * z, axis=1).reshape(G, 1, _T)
    w2 = jnp.sum(W * W, axis=1).reshape(1, _NV)
    idx3 = pl.pallas_call(
        _argmin_body,
        grid=(G,),
        in_specs=[
            pl.BlockSpec((_T, D), lambda i: (i, 0)),
            pl.BlockSpec((_NV, D), lambda i: (0, 0)),
            pl.BlockSpec((1, 1, _T), lambda i: (i, 0, 0)),
            pl.BlockSpec((1, _NV), lambda i: (0, 0)),
        ],
        out_specs=pl.BlockSpec((1, 1, _T), lambda i: (i, 0, 0)),
        out_shape=jax.ShapeDtypeStruct((G, 1, _T), jnp.int32),
    )(z, W, z2, w2)
    idx = idx3.reshape(N)

    w_pad = jnp.pad(W, ((0, 0), (0, 128 - D)))
    mesh = plsc.VectorSubcoreMesh(core_axis_name="c", subcore_axis_name="s")
    zq_flat, partials = pl.kernel(
        _sc_gather_loss,
        mesh=mesh,
        out_type=[
            jax.ShapeDtypeStruct((N, D), jnp.float32),
            jax.ShapeDtypeStruct((_NW, _L), jnp.float32),
        ],
        scratch_types=[
            pltpu.VMEM((_BPW,), jnp.int32),
            pltpu.VMEM((_BPW, 128), jnp.float32),
            pltpu.VMEM((_BPW, _D), jnp.float32),
            pltpu.VMEM((_BPW, _D), jnp.float32),
            pltpu.VMEM((_L,), jnp.float32),
            pltpu.SemaphoreType.DMA,
        ],
    )(w_pad, idx, inp2)

    z_q = zq_flat.reshape(input.shape)
    loss = jnp.sum(partials) * ((1.0 + _BETA) / (N * D))
    return (z_q, idx, loss)


# speedup vs baseline: 1.3285x; 1.0395x over previous
import jax
import jax.numpy as jnp
from jax import lax
from jax.experimental import pallas as pl
from jax.experimental.pallas import tpu as pltpu, tpu_sc as plsc

_NV = 8192
_D = 32
_BETA = 0.25
_T = 1024
_WIN = 2048
_N = 8192

_info = plsc.get_sparse_core_info()
_NC, _NS, _L = _info.num_cores, _info.num_subcores, _info.num_lanes
_NW = _NC * _NS
_BPW = _N // _NW


def _argmin_body(z_ref, w_ref, z2_ref, w2_ref, idx_ref):
    zt = z_ref[...]
    w = w_ref[...]
    z2 = z2_ref[0, 0, :]
    w2 = w2_ref[0, :]
    dot = jax.lax.dot_general(zt.astype(jnp.bfloat16), w.astype(jnp.bfloat16),
                              (((1,), (1,)), ((), ())),
                              preferred_element_type=jnp.float32)
    s = (z2[:, None] + w2[None, :]) - 2.0 * dot
    d = jnp.sqrt(jnp.maximum(s, 0.0))

    nw = _NV // _WIN
    best_v = jnp.full((_T,), jnp.inf, jnp.float32)
    best_i = jnp.zeros((_T,), jnp.int32)
    col = jax.lax.broadcasted_iota(jnp.int32, (_T, _WIN), 1)
    for wi in range(nw):
        dw = d[:, wi * _WIN:(wi + 1) * _WIN]
        mv = jnp.min(dw, axis=1)
        li = jnp.min(jnp.where(dw == mv[:, None], col, _WIN), axis=1) + wi * _WIN
        take = mv < best_v
        mb = jax.lax.bitcast_convert_type(mv, jnp.uint32)
        mb = (mb + 0x7FFF + ((mb >> 16) & 1)) & jnp.uint32(0xFFFF0000)
        mvr = jax.lax.bitcast_convert_type(mb, jnp.float32)
        best_v = jnp.where(take, mvr, best_v)
        best_i = jnp.where(take, li, best_i)
    idx_ref[0, 0, :] = best_i


def _sc_gather_loss(w_hbm, idx_hbm, inp_hbm, zq_hbm, part_hbm,
                    idx_v, rows_v, inp_v, zqc_v, part_v, sem):
    wid = lax.axis_index("s") * _NC + lax.axis_index("c")
    base = wid * _BPW
    pltpu.sync_copy(idx_hbm.at[pl.ds(base, _BPW)], idx_v)
    pltpu.async_copy(w_hbm.at[idx_v], rows_v, sem).wait()
    pltpu.sync_copy(inp_hbm.at[pl.ds(base, _BPW)], inp_v)

    def body(t, acc):
        a0 = rows_v[t, pl.ds(0, _L)]
        a1 = rows_v[t, pl.ds(_L, _L)]
        zqc_v[t, pl.ds(0, _L)] = a0
        zqc_v[t, pl.ds(_L, _L)] = a1
        e0 = a0 - inp_v[t, pl.ds(0, _L)]
        e1 = a1 - inp_v[t, pl.ds(_L, _L)]
        return acc + e0 * e0 + e1 * e1

    acc = lax.fori_loop(0, _BPW, body, jnp.zeros((_L,), jnp.float32))
    pltpu.sync_copy(zqc_v, zq_hbm.at[pl.ds(base, _BPW)])
    part_v[...] = acc
    pltpu.sync_copy(part_v, part_hbm.at[wid])


def kernel(input, W):
    B, D, H, Wd = input.shape
    N = B * H * Wd
    z = jnp.transpose(input, (0, 2, 3, 1)).reshape(N, D)
    inp2 = input.reshape(N, D)
    G = N // _T
    z2 = jnp.sum(z * z, axis=1).reshape(G, 1, _T)
    w2 = jnp.sum(W * W, axis=1).reshape(1, _NV)
    idx3 = pl.pallas_call(
        _argmin_body,
        grid=(G,),
        in_specs=[
            pl.BlockSpec((_T, D), lambda i: (i, 0)),
            pl.BlockSpec((_NV, D), lambda i: (0, 0)),
            pl.BlockSpec((1, 1, _T), lambda i: (i, 0, 0)),
            pl.BlockSpec((1, _NV), lambda i: (0, 0)),
        ],
        out_specs=pl.BlockSpec((1, 1, _T), lambda i: (i, 0, 0)),
        out_shape=jax.ShapeDtypeStruct((G, 1, _T), jnp.int32),
    )(z, W, z2, w2)
    idx = idx3.reshape(N)

    w_pad = jnp.pad(W, ((0, 0), (0, 128 - D)))
    mesh = plsc.VectorSubcoreMesh(core_axis_name="c", subcore_axis_name="s")
    zq_flat, partials = pl.kernel(
        _sc_gather_loss,
        mesh=mesh,
        out_type=[
            jax.ShapeDtypeStruct((N, D), jnp.float32),
            jax.ShapeDtypeStruct((_NW, _L), jnp.float32),
        ],
        scratch_types=[
            pltpu.VMEM((_BPW,), jnp.int32),
            pltpu.VMEM((_BPW, 128), jnp.float32),
            pltpu.VMEM((_BPW, _D), jnp.float32),
            pltpu.VMEM((_BPW, _D), jnp.float32),
            pltpu.VMEM((_L,), jnp.float32),
            pltpu.SemaphoreType.DMA,
        ],
    )(w_pad, idx, inp2)

    z_q = zq_flat.reshape(input.shape)
    loss = jnp.sum(partials) * ((1.0 + _BETA) / (N * D))
    return (z_q, idx, loss)
